# Initial kernel scaffold; baseline (speedup 1.0000x reference)
import jax
import jax.numpy as jnp
from jax.experimental import pallas as pl


def kernel(x, W):
    raise NotImplementedError("write your pallas kernel here")


# speedup vs baseline: 1.1969x; 1.1969x over previous
import functools

import jax
import jax.numpy as jnp
from jax import lax
from jax.experimental import pallas as pl
from jax.experimental.pallas import tpu as pltpu
from jax.experimental.pallas import tpu_sc as plsc

_EMB = 256
_K = 1024
_M = 256


def _dist_argmin_body(xf_ref, w_ref, b2_ref, idx_ref):
    xf = xf_ref[...]
    a2 = jnp.sum(xf * xf, axis=1, keepdims=True)
    mm = lax.dot_general(xf, w_ref[...], (((1,), (1,)), ((), ())),
                         preferred_element_type=jnp.float32)
    dist = jnp.sqrt(jnp.maximum((a2 + b2_ref[...]) - 2.0 * mm, 0.0))
    mv = jnp.min(dist, axis=1, keepdims=True)
    ks = lax.broadcasted_iota(jnp.int32, dist.shape, 1)
    idx = jnp.min(jnp.where(dist == mv, ks, _K), axis=1)
    idx_ref[...] = idx[None, None, :].astype(jnp.int32)


def _tc_indices(xf, W, b2):
    n = xf.shape[0]
    nb = n // _M
    return pl.pallas_call(
        _dist_argmin_body,
        grid=(nb,),
        in_specs=[
            pl.BlockSpec((_M, _EMB), lambda i: (i, 0)),
            pl.BlockSpec((_K, _EMB), lambda i: (0, 0)),
            pl.BlockSpec((1, _K), lambda i: (0, 0)),
        ],
        out_specs=pl.BlockSpec((1, 1, _M), lambda i: (i, 0, 0)),
        out_shape=jax.ShapeDtypeStruct((nb, 1, _M), jnp.int32),
        compiler_params=pltpu.CompilerParams(
            dimension_semantics=("arbitrary",)),
    )(xf, W, b2)


_NC, _NS = 2, 16
_NW = _NC * _NS
_N = 16384
_BPW = _N // _NW
_CH = 256
_NCHUNK = _BPW // _CH


def _make_sc_gather():
    @functools.partial(
        pl.kernel,
        mesh=plsc.VectorSubcoreMesh(core_axis_name="c", subcore_axis_name="s"),
        out_type=jax.ShapeDtypeStruct((_N, _EMB), jnp.float32),
        scratch_types=[
            pltpu.VMEM((_CH,), jnp.int32),
            pltpu.VMEM((_CH, _EMB), jnp.float32),
            pltpu.SemaphoreType.DMA,
        ],
    )
    def _sc_gather(table_hbm, idx_hbm, out_hbm, idx_v, rows_v, sem):
        wid = lax.axis_index("s") * _NC + lax.axis_index("c")
        base = wid * _BPW
        for ci in range(_NCHUNK):
            pltpu.sync_copy(idx_hbm.at[pl.ds(base + ci * _CH, _CH)], idx_v)
            pltpu.async_copy(table_hbm.at[idx_v], rows_v, sem).wait()
            pltpu.sync_copy(rows_v, out_hbm.at[pl.ds(base + ci * _CH, _CH)])

    return _sc_gather


def kernel(x, W):
    b, c, h, w = x.shape
    n = b * h * w
    xf = jnp.transpose(x, (0, 2, 3, 1)).reshape(n, c)
    b2 = jnp.sum(W * W, axis=1)[None, :]
    idx = _tc_indices(xf, W, b2).reshape(n)
    qf = _make_sc_gather()(W, idx)
    quantized = jnp.transpose(qf.reshape(b, h, w, c), (0, 3, 1, 2))
    return (quantized, idx)


# speedup vs baseline: 1.2426x; 1.0382x over previous
import functools

import jax
import jax.numpy as jnp
from jax import lax
from jax.experimental import pallas as pl
from jax.experimental.pallas import tpu as pltpu
from jax.experimental.pallas import tpu_sc as plsc

_EMB = 256
_K = 1024
_M = 512


def _dist_argmin_body(xf_ref, w2_ref, b2_ref, idx_ref):
    xf = xf_ref[...]
    a2 = jnp.sum(xf * xf, axis=1, keepdims=True)
    mm2 = lax.dot_general(xf, w2_ref[...], (((1,), (1,)), ((), ())),
                          preferred_element_type=jnp.float32)
    dist = jnp.sqrt(jnp.maximum((a2 + b2_ref[...]) - mm2, 0.0))
    mv = jnp.min(dist, axis=1, keepdims=True)
    ks = lax.broadcasted_iota(jnp.int32, dist.shape, 1)
    idx = jnp.min(jnp.where(dist == mv, ks, _K), axis=1)
    idx_ref[...] = idx[None, None, :].astype(jnp.int32)


def _tc_indices(xf, W2, b2):
    n = xf.shape[0]
    nb = n // _M
    return pl.pallas_call(
        _dist_argmin_body,
        grid=(nb,),
        in_specs=[
            pl.BlockSpec((_M, _EMB), lambda i: (i, 0)),
            pl.BlockSpec((_K, _EMB), lambda i: (0, 0)),
            pl.BlockSpec((1, _K), lambda i: (0, 0)),
        ],
        out_specs=pl.BlockSpec((1, 1, _M), lambda i: (i, 0, 0)),
        out_shape=jax.ShapeDtypeStruct((nb, 1, _M), jnp.int32),
        compiler_params=pltpu.CompilerParams(
            dimension_semantics=("arbitrary",)),
    )(xf, W2, b2)


_NC, _NS = 2, 16
_NW = _NC * _NS
_N = 16384
_BPW = _N // _NW
_CH = 128
_NCHUNK = _BPW // _CH


def _make_sc_gather():
    @functools.partial(
        pl.kernel,
        mesh=plsc.VectorSubcoreMesh(core_axis_name="c", subcore_axis_name="s"),
        out_type=jax.ShapeDtypeStruct((_N, _EMB), jnp.float32),
        scratch_types=[
            pltpu.VMEM((_CH,), jnp.int32),
            pltpu.VMEM((_CH,), jnp.int32),
            pltpu.VMEM((_CH, _EMB), jnp.float32),
            pltpu.VMEM((_CH, _EMB), jnp.float32),
            pltpu.SemaphoreType.DMA,
            pltpu.SemaphoreType.DMA,
            pltpu.SemaphoreType.DMA,
            pltpu.SemaphoreType.DMA,
        ],
    )
    def _sc_gather(table_hbm, idx_hbm, out_hbm, idx_v0, idx_v1, rows_v0,
                   rows_v1, sg0, sg1, sw0, sw1):
        wid = lax.axis_index("s") * _NC + lax.axis_index("c")
        base = wid * _BPW
        idx_v = (idx_v0, idx_v1)
        rows_v = (rows_v0, rows_v1)
        sg = (sg0, sg1)
        sw = (sw0, sw1)
        gathers = [None] * _NCHUNK
        writes = [None] * _NCHUNK
        pltpu.sync_copy(idx_hbm.at[pl.ds(base, _CH)], idx_v0)
        gathers[0] = pltpu.async_copy(table_hbm.at[idx_v0], rows_v0, sg0)
        for ci in range(_NCHUNK):
            p = ci % 2
            if ci + 1 < _NCHUNK:
                q = (ci + 1) % 2
                pltpu.sync_copy(
                    idx_hbm.at[pl.ds(base + (ci + 1) * _CH, _CH)], idx_v[q])
                if ci >= 1:
                    writes[ci - 1].wait()
                gathers[ci + 1] = pltpu.async_copy(
                    table_hbm.at[idx_v[q]], rows_v[q], sg[q])
            gathers[ci].wait()
            writes[ci] = pltpu.async_copy(
                rows_v[p], out_hbm.at[pl.ds(base + ci * _CH, _CH)], sw[p])
        writes[_NCHUNK - 2].wait()
        writes[_NCHUNK - 1].wait()

    return _sc_gather


def kernel(x, W):
    b, c, h, w = x.shape
    n = b * h * w
    xf = jnp.transpose(x, (0, 2, 3, 1)).reshape(n, c)
    b2 = jnp.sum(W * W, axis=1)[None, :]
    idx = _tc_indices(xf, 2.0 * W, b2).reshape(n)
    qf = _make_sc_gather()(W, idx)
    quantized = jnp.transpose(qf.reshape(b, h, w, c), (0, 3, 1, 2))
    return (quantized, idx)
